# baseline (speedup 1.0000x reference)
import jax
import jax.numpy as jnp
from jax import lax
from jax.experimental import pallas as pl
from jax.experimental.pallas import tpu as pltpu
from jax.experimental.pallas import tpu_sc as plsc

N = 10000
E = 320000
D = 128
R = 8
R1 = R + 1
DH = D // 2

NC = 2
NS = 16
CH = 80
NP = 10112
EXT = E + NP + 2688
ROWS = EXT // CH
RPTILE = ROWS // NS
EPB = 65
NBLK = RPTILE // EPB
NSLOT = 5
RPT = NP // NS
ZR = 128
WBS = (128, 128, 128, 128, 120)
WBO = (0, 128, 256, 384, 512)
NWB = len(WBS)


def _mm_body(x_ref, w_ref, o_ref):
    x = x_ref[...]
    for r in range(R1):
        res = jnp.dot(x, w_ref[r], preferred_element_type=jnp.float32)
        o_ref[0, r] = res[:, :DH]
        o_ref[1, r] = res[:, DH:]


def _relation_matmuls(xp, Wf):
    blk = 1264
    nb = NP // blk
    return pl.pallas_call(
        _mm_body,
        grid=(nb,),
        in_specs=[
            pl.BlockSpec((blk, D), lambda i: (i, 0)),
            pl.BlockSpec((R1, D, D), lambda i: (0, 0, 0)),
        ],
        out_specs=pl.BlockSpec((NC, R1, blk, DH), lambda i: (0, 0, i, 0)),
        out_shape=jax.ShapeDtypeStruct((NC, R1, NP, DH), jnp.float32),
    )(xp, Wf)


def _sc_body(xw_hbm, src_hbm, dst_hbm, typ_hbm, out_hbm,
             acc_sh, deg_sh, zbuf, zdbuf, src2_v, dst2_v, typ2_v, g2_v,
             rows_v, ones_v,
             sg0, sg1, sg2, sg3, sg4, ss0, ss1, ss2, ss3, ss4,
             sem_d, sem_e, sem_r, sem_w):
    sem_g = (sg0, sg1, sg2, sg3, sg4)
    sem_s = (ss0, ss1, ss2, ss3, ss4)
    c = lax.axis_index("c")
    s = lax.axis_index("s")
    rbase = s * RPT
    goff = c * (R1 * NP)

    zero16 = jnp.zeros((16,), jnp.float32)
    one16 = jnp.ones((16,), jnp.float32)

    def _z(i, _):
        for q in range(DH // 16):
            zbuf[0, i, pl.ds(q * 16, 16)] = zero16
        zdbuf[0, i, :] = zero16
        return 0
    lax.fori_loop(0, ZR, _z, 0)

    def _o(i, _):
        ones_v[i, :] = one16
        return 0
    lax.fori_loop(0, CH, _o, 0)

    for k in range(NWB):
        off, sz = rbase + WBO[k], WBS[k]
        pltpu.async_copy(zbuf.at[0].at[pl.ds(0, sz)],
                         acc_sh.at[pl.ds(off, sz)], sem_r)
        pltpu.async_copy(zdbuf.at[0].at[pl.ds(0, sz)],
                         deg_sh.at[pl.ds(off, sz)], sem_r)
    for k in range(NWB):
        sz = WBS[k]
        pltpu.make_async_copy(zbuf.at[0].at[pl.ds(0, sz)],
                              acc_sh.at[pl.ds(rbase, sz)], sem_r).wait()
        pltpu.make_async_copy(zdbuf.at[0].at[pl.ds(0, sz)],
                              deg_sh.at[pl.ds(rbase, sz)], sem_r).wait()

    plsc.subcore_barrier()

    def _fire_g(j, b):
        pltpu.async_copy(xw_hbm.at[g2_v.at[j]], rows_v.at[b], sem_g[b])

    def _wait_g(b):
        pltpu.make_async_copy(xw_hbm.at[g2_v.at[0]], rows_v.at[b],
                              sem_g[b]).wait()

    def _wait_s(b):
        pltpu.make_async_copy(rows_v.at[b], acc_sh.at[dst2_v.at[0, 0]],
                              sem_s[b]).wait()

    def _wait_d():
        pltpu.make_async_copy(ones_v, deg_sh.at[dst2_v.at[0, 0]], sem_d).wait()

    def _fire_edges(bset, k):
        rowb = s * RPTILE + k * EPB
        pltpu.async_copy(src_hbm.at[pl.ds(rowb, EPB)], src2_v.at[bset], sem_e)
        pltpu.async_copy(dst_hbm.at[pl.ds(rowb, EPB)], dst2_v.at[bset], sem_e)
        pltpu.async_copy(typ_hbm.at[pl.ds(rowb, EPB)], typ2_v.at[bset], sem_e)

    def _wait_edges(bset):
        for ref in (src2_v, dst2_v, typ2_v):
            pltpu.make_async_copy(src_hbm.at[pl.ds(0, EPB)], ref.at[bset],
                                  sem_e).wait()

    for k in range(NBLK):
        bset = 0
        _fire_edges(bset, k)
        _wait_edges(bset)

        def _gidx(r, _):
            for i in range(CH // 16):
                sl = pl.ds(i * 16, 16)
                g2_v[r, sl] = (goff + typ2_v[bset, r, sl] * NP
                               + src2_v[bset, r, sl])
            return 0
        lax.fori_loop(0, EPB, _gidx, 0)

        for b in range(3):
            _fire_g(b, b)

        def _step(gg, _):
            for b in range(NSLOT):
                j = gg * NSLOT + b
                _wait_g(b)
                pltpu.async_copy(rows_v.at[b], acc_sh.at[dst2_v.at[bset, j]],
                                 sem_s[b], add=True)
                pltpu.async_copy(ones_v, deg_sh.at[dst2_v.at[bset, j]], sem_d,
                                 add=True)

                @pl.when(j >= 2)
                def _():
                    _wait_s((b + 3) % NSLOT)

                @pl.when(j + 3 < EPB)
                def _():
                    _fire_g(j + 3, (b + 3) % NSLOT)
            return 0
        lax.fori_loop(0, EPB // NSLOT, _step, 0)

        _wait_s((EPB - 2) % NSLOT)
        _wait_s((EPB - 1) % NSLOT)

        def _dd(i, _):
            _wait_d()
            return 0
        lax.fori_loop(0, EPB, _dd, 0)

    plsc.subcore_barrier()

    def _stage(wb, k):
        off, sz = rbase + WBO[k], WBS[k]
        pltpu.async_copy(acc_sh.at[pl.ds(off, sz)],
                         zbuf.at[wb].at[pl.ds(0, sz)], sem_r)
        pltpu.async_copy(deg_sh.at[pl.ds(off, sz)],
                         zdbuf.at[wb].at[pl.ds(0, sz)], sem_r)

    def _drain_stage(wb, k):
        sz = WBS[k]
        pltpu.make_async_copy(acc_sh.at[pl.ds(rbase, sz)],
                              zbuf.at[wb].at[pl.ds(0, sz)], sem_r).wait()
        pltpu.make_async_copy(deg_sh.at[pl.ds(rbase, sz)],
                              zdbuf.at[wb].at[pl.ds(0, sz)], sem_r).wait()

    def _drain_write(wb, k):
        sz = WBS[k]
        pltpu.make_async_copy(zbuf.at[wb].at[pl.ds(0, sz)],
                              out_hbm.at[c, pl.ds(rbase, sz)], sem_w).wait()

    _stage(0, 0)
    for k in range(NWB):
        wb = k % 2
        _drain_stage(wb, k)
        if k >= 1:
            _drain_write(1 - wb, k - 1)
        if k + 1 < NWB:
            _stage(1 - wb, k + 1)

        def _div(r, _):
            dvec = jnp.maximum(zdbuf[wb, r, :] - 1.0, 1.0)
            for i in range(DH // 16):
                sl = pl.ds(i * 16, 16)
                zbuf[wb, r, sl] = zbuf[wb, r, sl] / dvec
            return 0
        lax.fori_loop(0, WBS[k], _div, 0)
        pltpu.async_copy(zbuf.at[wb].at[pl.ds(0, WBS[k])],
                         out_hbm.at[c, pl.ds(rbase + WBO[k], WBS[k])], sem_w)
    _drain_write((NWB - 1) % 2, NWB - 1)


def _sc_aggregate(xw, src2, dst2, typ2):
    mesh = plsc.VectorSubcoreMesh(core_axis_name="c", subcore_axis_name="s")
    f = pl.kernel(
        _sc_body,
        out_type=jax.ShapeDtypeStruct((NC, NP, DH), jnp.float32),
        mesh=mesh,
        compiler_params=pltpu.CompilerParams(use_tc_tiling_on_sc=False),
        scratch_types=[
            pltpu.VMEM_SHARED((NP, DH), jnp.float32),
            pltpu.VMEM_SHARED((NP, 16), jnp.float32),
            pltpu.VMEM((2, ZR, DH), jnp.float32),
            pltpu.VMEM((2, ZR, 16), jnp.float32),
            pltpu.VMEM((1, EPB, CH), jnp.int32),
            pltpu.VMEM((1, EPB, CH), jnp.int32),
            pltpu.VMEM((1, EPB, CH), jnp.int32),
            pltpu.VMEM((EPB, CH), jnp.int32),
            pltpu.VMEM((NSLOT, CH, DH), jnp.float32),
            pltpu.VMEM((CH, 16), jnp.float32),
            pltpu.SemaphoreType.DMA,
            pltpu.SemaphoreType.DMA,
            pltpu.SemaphoreType.DMA,
            pltpu.SemaphoreType.DMA,
            pltpu.SemaphoreType.DMA,
            pltpu.SemaphoreType.DMA,
            pltpu.SemaphoreType.DMA,
            pltpu.SemaphoreType.DMA,
            pltpu.SemaphoreType.DMA,
            pltpu.SemaphoreType.DMA,
            pltpu.SemaphoreType.DMA,
            pltpu.SemaphoreType.DMA,
            pltpu.SemaphoreType.DMA,
            pltpu.SemaphoreType.DMA,
        ],
    )
    return f(xw, src2, dst2, typ2)


@jax.jit
def _run(x, edge_index, edge_type, W, W0):
    Wf = jnp.concatenate([W, W0[None]], axis=0)
    xp = jnp.pad(x, ((0, NP - N), (0, 0)))
    xw = _relation_matmuls(xp, Wf).reshape(NC * R1 * NP, DH)
    nodes = jnp.arange(NP, dtype=jnp.int32)
    npad = EXT - E - NP
    src2 = jnp.concatenate(
        [edge_index[0], nodes, jnp.zeros((npad,), jnp.int32)]).reshape(ROWS, CH)
    absorber = N + (jnp.arange(npad, dtype=jnp.int32) % (NP - N))
    dst2 = jnp.concatenate(
        [edge_index[1], nodes, absorber]).reshape(ROWS, CH)
    typ2 = jnp.concatenate(
        [edge_type, jnp.full((NP,), R, jnp.int32),
         jnp.zeros((npad,), jnp.int32)]).reshape(ROWS, CH)
    halves = _sc_aggregate(xw, src2, dst2, typ2)
    return jnp.concatenate([halves[0, :N], halves[1, :N]], axis=1)


def kernel(x, edge_index, edge_type, num_nodes, W, W0):
    return _run(x, edge_index, edge_type, W, W0)


# speedup vs baseline: 1.0738x; 1.0738x over previous
import jax
import jax.numpy as jnp
from jax import lax
from jax.experimental import pallas as pl
from jax.experimental.pallas import tpu as pltpu
from jax.experimental.pallas import tpu_sc as plsc

N = 10000
E = 320000
D = 128
R = 8
R1 = R + 1
DH = D // 2

NC = 2
NS = 16
CH = 80
ROWS = E // CH
RPTILE = ROWS // NS
EPB = 50
NBLK = RPTILE // EPB
NSLOT = 5
DT = 72
NP = 10240
RPT = NP // NS
ZR = 128


def _mm_body(x_ref, w_ref, o_ref):
    x = x_ref[...]
    blk = x.shape[0]
    aux = jnp.concatenate(
        [jnp.ones((blk, 1), jnp.float32), jnp.zeros((blk, 7), jnp.float32)],
        axis=1)
    for r in range(R1):
        res = jnp.dot(x, w_ref[r], preferred_element_type=jnp.float32)
        o_ref[0, r] = jnp.concatenate([res[:, :DH], aux], axis=1)
        o_ref[1, r] = jnp.concatenate([res[:, DH:], aux], axis=1)


def _relation_matmuls(xp, Wf):
    blk = 2048
    nb = NP // blk
    return pl.pallas_call(
        _mm_body,
        grid=(nb,),
        in_specs=[
            pl.BlockSpec((blk, D), lambda i: (i, 0)),
            pl.BlockSpec((R1, D, D), lambda i: (0, 0, 0)),
        ],
        out_specs=pl.BlockSpec((NC, R1, blk, DT), lambda i: (0, 0, i, 0)),
        out_shape=jax.ShapeDtypeStruct((NC, R1, NP, DT), jnp.float32),
    )(xp, Wf)


def _sc_body(xw_hbm, src_hbm, dst_hbm, typ_hbm, out_hbm,
             acc_sh, zbuf, src2_v, dst2_v, typ2_v, g2_v,
             rows_v,
             sg0, sg1, sg2, sg3, sg4, ss0, ss1, ss2, ss3, ss4):
    sem_g = (sg0, sg1, sg2, sg3, sg4)
    sem_s = (ss0, ss1, ss2, ss3, ss4)
    c = lax.axis_index("c")
    s = lax.axis_index("s")
    rbase = s * RPT
    goff = c * (R1 * NP)
    w0off = goff + R * NP

    for k in range(RPT // ZR):
        off = rbase + k * ZR
        pltpu.sync_copy(xw_hbm.at[pl.ds(w0off + off, ZR)], zbuf)
        pltpu.sync_copy(zbuf, acc_sh.at[pl.ds(off, ZR)])

    plsc.subcore_barrier()

    def _fire_g(j, b):
        pltpu.async_copy(xw_hbm.at[g2_v.at[j]], rows_v.at[b], sem_g[b])

    def _wait_g(b):
        pltpu.make_async_copy(xw_hbm.at[g2_v.at[0]], rows_v.at[b],
                              sem_g[b]).wait()

    def _wait_s(b):
        pltpu.make_async_copy(rows_v.at[b], acc_sh.at[dst2_v.at[0]],
                              sem_s[b]).wait()

    def _block(k, _):
        rowb = s * RPTILE + k * EPB
        pltpu.sync_copy(src_hbm.at[pl.ds(rowb, EPB)], src2_v)
        pltpu.sync_copy(dst_hbm.at[pl.ds(rowb, EPB)], dst2_v)
        pltpu.sync_copy(typ_hbm.at[pl.ds(rowb, EPB)], typ2_v)

        def _gidx(r, _):
            for i in range(CH // 16):
                sl = pl.ds(i * 16, 16)
                g2_v[r, sl] = goff + typ2_v[r, sl] * NP + src2_v[r, sl]
            return 0
        lax.fori_loop(0, EPB, _gidx, 0)

        for b in range(3):
            _fire_g(b, b)

        def _step(gg, _):
            for b in range(NSLOT):
                j = gg * NSLOT + b
                _wait_g(b)
                pltpu.async_copy(rows_v.at[b], acc_sh.at[dst2_v.at[j]],
                                 sem_s[b], add=True)

                @pl.when(j >= 2)
                def _():
                    _wait_s((b + 3) % NSLOT)

                @pl.when(j + 3 < EPB)
                def _():
                    _fire_g(j + 3, (b + 3) % NSLOT)
            return 0
        lax.fori_loop(0, EPB // NSLOT, _step, 0)

        _wait_s((EPB - 2) % NSLOT)
        _wait_s((EPB - 1) % NSLOT)
        return 0
    lax.fori_loop(0, NBLK, _block, 0)

    plsc.subcore_barrier()

    for k in range(RPT // ZR):
        pltpu.sync_copy(acc_sh.at[pl.ds(rbase + k * ZR, ZR)], zbuf)

        def _div(r, _):
            v = zbuf[r, pl.ds(DH - 8, 16)]
            dr = jnp.maximum(v[8] - 1.0, 1.0)
            for i in range(DH // 16):
                sl = pl.ds(i * 16, 16)
                zbuf[r, sl] = zbuf[r, sl] / dr
            return 0
        lax.fori_loop(0, ZR, _div, 0)
        pltpu.sync_copy(zbuf, out_hbm.at[c, pl.ds(rbase + k * ZR, ZR)])


def _sc_aggregate(xw, src2, dst2, typ2):
    mesh = plsc.VectorSubcoreMesh(core_axis_name="c", subcore_axis_name="s")
    f = pl.kernel(
        _sc_body,
        out_type=jax.ShapeDtypeStruct((NC, NP, DT), jnp.float32),
        mesh=mesh,
        compiler_params=pltpu.CompilerParams(use_tc_tiling_on_sc=False),
        scratch_types=[
            pltpu.VMEM_SHARED((NP, DT), jnp.float32),
            pltpu.VMEM((ZR, DT), jnp.float32),
            pltpu.VMEM((EPB, CH), jnp.int32),
            pltpu.VMEM((EPB, CH), jnp.int32),
            pltpu.VMEM((EPB, CH), jnp.int32),
            pltpu.VMEM((EPB, CH), jnp.int32),
            pltpu.VMEM((NSLOT, CH, DT), jnp.float32),
            pltpu.SemaphoreType.DMA,
            pltpu.SemaphoreType.DMA,
            pltpu.SemaphoreType.DMA,
            pltpu.SemaphoreType.DMA,
            pltpu.SemaphoreType.DMA,
            pltpu.SemaphoreType.DMA,
            pltpu.SemaphoreType.DMA,
            pltpu.SemaphoreType.DMA,
            pltpu.SemaphoreType.DMA,
            pltpu.SemaphoreType.DMA,
        ],
    )
    return f(xw, src2, dst2, typ2)


@jax.jit
def _run(x, edge_index, edge_type, W, W0):
    Wf = jnp.concatenate([W, W0[None]], axis=0)
    xp = jnp.pad(x, ((0, NP - N), (0, 0)))
    xw = _relation_matmuls(xp, Wf).reshape(NC * R1 * NP, DT)
    src2 = edge_index[0].reshape(ROWS, CH)
    dst2 = edge_index[1].reshape(ROWS, CH)
    typ2 = edge_type.reshape(ROWS, CH)
    halves = _sc_aggregate(xw, src2, dst2, typ2)
    return jnp.concatenate([halves[0, :N, :DH], halves[1, :N, :DH]], axis=1)


def kernel(x, edge_index, edge_type, num_nodes, W, W0):
    return _run(x, edge_index, edge_type, W, W0)


# speedup vs baseline: 1.2293x; 1.1448x over previous
import jax
import jax.numpy as jnp
from jax import lax
from jax.experimental import pallas as pl
from jax.experimental.pallas import tpu as pltpu
from jax.experimental.pallas import tpu_sc as plsc

N = 10000
E = 320000
D = 128
R = 8
R1 = R + 1
DH = D // 2

NC = 2
NS = 16
CH = 80
ROWS = E // CH
RPTILE = ROWS // NS
EPB = 50
NBLK = RPTILE // EPB
NSLOT = 5
NP = 10240
RPT = NP // NS
ZR = 128


def _mm_body(x_ref, w_ref, o_ref):
    x = x_ref[...]
    for r in range(R1):
        res = jnp.dot(x, w_ref[r], preferred_element_type=jnp.float32)
        o_ref[0, r] = res[:, :DH]
        o_ref[1, r] = res[:, DH:]


def _relation_matmuls(xp, Wf):
    blk = 2048
    nb = NP // blk
    return pl.pallas_call(
        _mm_body,
        grid=(nb,),
        in_specs=[
            pl.BlockSpec((blk, D), lambda i: (i, 0)),
            pl.BlockSpec((R1, D, D), lambda i: (0, 0, 0)),
        ],
        out_specs=pl.BlockSpec((NC, R1, blk, DH), lambda i: (0, 0, i, 0)),
        out_shape=jax.ShapeDtypeStruct((NC, R1, NP, DH), jnp.float32),
    )(xp, Wf)


def _sc_body(xw_hbm, src_hbm, dst_hbm, typ_hbm, out_hbm,
             acc_sh, deg_sh, zbuf, zdbuf, src2_v, dst2_v, typ2_v, g2_v,
             rows_v, ones_v,
             sg0, sg1, sg2, sg3, sg4, ss0, ss1, ss2, ss3, ss4, sem_d):
    sem_g = (sg0, sg1, sg2, sg3, sg4)
    sem_s = (ss0, ss1, ss2, ss3, ss4)
    c = lax.axis_index("c")
    s = lax.axis_index("s")
    rbase = s * RPT
    goff = c * (R1 * NP)
    w0off = goff + R * NP

    zero16 = jnp.zeros((16,), jnp.float32)
    one16 = jnp.ones((16,), jnp.float32)

    for k in range(RPT // ZR):
        off = rbase + k * ZR
        pltpu.sync_copy(xw_hbm.at[pl.ds(w0off + off, ZR)], zbuf)
        pltpu.sync_copy(zbuf, acc_sh.at[pl.ds(off, ZR)])

    def _zd(i, _):
        zdbuf[i, :] = zero16
        return 0
    lax.fori_loop(0, ZR, _zd, 0)

    def _o(i, _):
        ones_v[i, :] = one16
        return 0
    lax.fori_loop(0, CH, _o, 0)

    for k in range(RPT // ZR):
        pltpu.sync_copy(zdbuf, deg_sh.at[pl.ds(rbase + k * ZR, ZR)])

    plsc.subcore_barrier()

    def _fire_g(j, b):
        pltpu.async_copy(xw_hbm.at[g2_v.at[j]], rows_v.at[b], sem_g[b])

    def _wait_g(b):
        pltpu.make_async_copy(xw_hbm.at[g2_v.at[0]], rows_v.at[b],
                              sem_g[b]).wait()

    def _wait_s(b):
        pltpu.make_async_copy(rows_v.at[b], acc_sh.at[dst2_v.at[0]],
                              sem_s[b]).wait()

    def _wait_d():
        pltpu.make_async_copy(ones_v, deg_sh.at[dst2_v.at[0]], sem_d).wait()

    def _block(k, _):
        rowb = s * RPTILE + k * EPB
        pltpu.sync_copy(src_hbm.at[pl.ds(rowb, EPB)], src2_v)
        pltpu.sync_copy(dst_hbm.at[pl.ds(rowb, EPB)], dst2_v)
        pltpu.sync_copy(typ_hbm.at[pl.ds(rowb, EPB)], typ2_v)

        def _gidx(r, _):
            for i in range(CH // 16):
                sl = pl.ds(i * 16, 16)
                g2_v[r, sl] = goff + typ2_v[r, sl] * NP + src2_v[r, sl]
            return 0
        lax.fori_loop(0, EPB, _gidx, 0)

        for b in range(3):
            _fire_g(b, b)

        def _step(gg, _):
            for b in range(NSLOT):
                j = gg * NSLOT + b
                _wait_g(b)
                pltpu.async_copy(rows_v.at[b], acc_sh.at[dst2_v.at[j]],
                                 sem_s[b], add=True)
                pltpu.async_copy(ones_v, deg_sh.at[dst2_v.at[j]], sem_d,
                                 add=True)

                @pl.when(j >= 2)
                def _():
                    _wait_s((b + 3) % NSLOT)

                @pl.when(j + 3 < EPB)
                def _():
                    _fire_g(j + 3, (b + 3) % NSLOT)
            return 0
        lax.fori_loop(0, EPB // NSLOT, _step, 0)

        _wait_s((EPB - 2) % NSLOT)
        _wait_s((EPB - 1) % NSLOT)

        def _dd(i, _):
            _wait_d()
            return 0
        lax.fori_loop(0, EPB, _dd, 0)
        return 0
    lax.fori_loop(0, NBLK, _block, 0)

    plsc.subcore_barrier()

    for k in range(RPT // ZR):
        pltpu.sync_copy(deg_sh.at[pl.ds(rbase + k * ZR, ZR)], zdbuf)
        pltpu.sync_copy(acc_sh.at[pl.ds(rbase + k * ZR, ZR)], zbuf)

        def _div(r, _):
            dvec = jnp.maximum(zdbuf[r, :], 1.0)
            for i in range(DH // 16):
                sl = pl.ds(i * 16, 16)
                zbuf[r, sl] = zbuf[r, sl] / dvec
            return 0
        lax.fori_loop(0, ZR, _div, 0)
        pltpu.sync_copy(zbuf, out_hbm.at[c, pl.ds(rbase + k * ZR, ZR)])


def _sc_aggregate(xw, src2, dst2, typ2):
    mesh = plsc.VectorSubcoreMesh(core_axis_name="c", subcore_axis_name="s")
    f = pl.kernel(
        _sc_body,
        out_type=jax.ShapeDtypeStruct((NC, NP, DH), jnp.float32),
        mesh=mesh,
        compiler_params=pltpu.CompilerParams(use_tc_tiling_on_sc=False),
        scratch_types=[
            pltpu.VMEM_SHARED((NP, DH), jnp.float32),
            pltpu.VMEM_SHARED((NP, 16), jnp.float32),
            pltpu.VMEM((ZR, DH), jnp.float32),
            pltpu.VMEM((ZR, 16), jnp.float32),
            pltpu.VMEM((EPB, CH), jnp.int32),
            pltpu.VMEM((EPB, CH), jnp.int32),
            pltpu.VMEM((EPB, CH), jnp.int32),
            pltpu.VMEM((EPB, CH), jnp.int32),
            pltpu.VMEM((NSLOT, CH, DH), jnp.float32),
            pltpu.VMEM((CH, 16), jnp.float32),
            pltpu.SemaphoreType.DMA,
            pltpu.SemaphoreType.DMA,
            pltpu.SemaphoreType.DMA,
            pltpu.SemaphoreType.DMA,
            pltpu.SemaphoreType.DMA,
            pltpu.SemaphoreType.DMA,
            pltpu.SemaphoreType.DMA,
            pltpu.SemaphoreType.DMA,
            pltpu.SemaphoreType.DMA,
            pltpu.SemaphoreType.DMA,
            pltpu.SemaphoreType.DMA,
        ],
    )
    return f(xw, src2, dst2, typ2)


@jax.jit
def _run(x, edge_index, edge_type, W, W0):
    Wf = jnp.concatenate([W, W0[None]], axis=0)
    xp = jnp.pad(x, ((0, NP - N), (0, 0)))
    xw = _relation_matmuls(xp, Wf).reshape(NC * R1 * NP, DH)
    src2 = edge_index[0].reshape(ROWS, CH)
    dst2 = edge_index[1].reshape(ROWS, CH)
    typ2 = edge_type.reshape(ROWS, CH)
    halves = _sc_aggregate(xw, src2, dst2, typ2)
    return jnp.concatenate([halves[0, :N], halves[1, :N]], axis=1)


def kernel(x, edge_index, edge_type, num_nodes, W, W0):
    return _run(x, edge_index, edge_type, W, W0)
